# baseline (speedup 1.0000x reference)
import functools

import jax
import jax.numpy as jnp
from jax import lax
from jax.experimental import pallas as pl
from jax.experimental.pallas import tpu as pltpu
from jax.experimental.pallas import tpu_sc as plsc

EMBED_DIM = 64
ROWS_PER_STREAM = 128
NBUF = 8


def _make_gather(num_workers, streams_per_worker, n_rows, dim):
  mesh = plsc.VectorSubcoreMesh(core_axis_name="c", subcore_axis_name="s")
  rows_per_worker = streams_per_worker * ROWS_PER_STREAM

  @functools.partial(
      pl.kernel,
      out_type=jax.ShapeDtypeStruct((n_rows, dim), jnp.float32),
      mesh=mesh,
      scratch_types=[
          pltpu.VMEM((streams_per_worker, ROWS_PER_STREAM), jnp.int32),
          pltpu.VMEM((NBUF, ROWS_PER_STREAM, dim), jnp.float32),
          pltpu.SemaphoreType.DMA,
      ],
      compiler_params=pltpu.CompilerParams(use_tc_tiling_on_sc=False),
  )
  def gather_kernel(idx_hbm, table_hbm, out_hbm, idx_v, rows_v, gsem):
    num_cores = jax.lax.axis_size("c")
    wid = lax.axis_index("s") * num_cores + lax.axis_index("c")
    base = wid * rows_per_worker

    pltpu.sync_copy(idx_hbm.at[wid], idx_v)

    def start_gather(j, buf):
      pltpu.make_async_copy(
          table_hbm.at[idx_v.at[j]], rows_v.at[buf], gsem).start()

    def finish_gather(j, buf):
      pltpu.make_async_copy(
          table_hbm.at[idx_v.at[j]], rows_v.at[buf], gsem).wait()
      pltpu.sync_copy(
          rows_v.at[buf],
          out_hbm.at[pl.ds(base + j * ROWS_PER_STREAM, ROWS_PER_STREAM)])

    for b in range(NBUF):
      start_gather(b, b)

    n_blocks = streams_per_worker // NBUF

    def body(i, carry):
      for b in range(NBUF):
        j = i * NBUF + b
        finish_gather(j, b)
        start_gather(j + NBUF, b)
      return carry

    lax.fori_loop(0, n_blocks - 1, body, 0)

    for b in range(NBUF):
      finish_gather((n_blocks - 1) * NBUF + b, b)

  return gather_kernel


def _mlp_body(e_ref, w1t_ref, b1_ref, w2_ref, b2_ref, o_ref):
  x2 = e_ref[...]
  d = w1t_ref.shape[0]
  half = pl.program_id(2)
  x = jnp.where(half == 0, x2[:, :d], x2[:, d:])
  h = jnp.dot(x, w1t_ref[...], preferred_element_type=jnp.float32)
  h = h + b1_ref[...]
  h = h * 0.5 * (1.0 + lax.erf(h * 0.7071067811865476))
  o = lax.dot_general(w2_ref[...], h, (((1,), (1,)), ((), ())),
                      preferred_element_type=jnp.float32)
  o_ref[0] = o + b2_ref[...]


def _mlp_t(e2, w1t, b1, w2, b2c, fields, batch, blk2):
  d = w2.shape[0]
  nb = (batch // 2) // blk2
  return pl.pallas_call(
      _mlp_body,
      grid=(fields, nb, 2),
      in_specs=[
          pl.BlockSpec((blk2, 2 * d), lambda f, j, h: (f * nb + j, 0)),
          pl.BlockSpec((d, d), lambda f, j, h: (0, 0)),
          pl.BlockSpec((1, d), lambda f, j, h: (0, 0)),
          pl.BlockSpec((d, d), lambda f, j, h: (0, 0)),
          pl.BlockSpec((d, 1), lambda f, j, h: (0, 0)),
      ],
      out_specs=pl.BlockSpec((1, d, blk2), lambda f, j, h: (f, 0, h * nb + j)),
      out_shape=jax.ShapeDtypeStruct((fields, d, batch), jnp.float32),
      compiler_params=pltpu.CompilerParams(
          dimension_semantics=("arbitrary", "arbitrary", "arbitrary")),
  )(e2, w1t, b1, w2, b2c)


def kernel(action_ids, W_emb, W1, b1, W2, b2):
  batch, fields = action_ids.shape
  n_rows = batch * fields
  dim = W_emb.shape[1]

  info = plsc.get_sparse_core_info()
  num_workers = info.num_cores * info.num_subcores
  streams_per_worker = n_rows // (num_workers * ROWS_PER_STREAM)

  idx = action_ids.T.reshape(fields, 2, batch // 2)
  idx = idx.transpose(0, 2, 1)
  idx3 = idx.reshape(num_workers, streams_per_worker, ROWS_PER_STREAM)
  e = _make_gather(num_workers, streams_per_worker, n_rows, dim)(idx3, W_emb)
  e2 = e.reshape(n_rows // 2, 2 * dim)
  out = _mlp_t(e2, W1.T, b1.reshape(1, dim), W2, b2.reshape(dim, 1),
               fields, batch, 1024)
  return out.transpose(2, 0, 1)


# speedup vs baseline: 1.1715x; 1.1715x over previous
import functools

import jax
import jax.numpy as jnp
from jax import lax
from jax.experimental import pallas as pl
from jax.experimental.pallas import tpu as pltpu
from jax.experimental.pallas import tpu_sc as plsc

EMBED_DIM = 64
ROWS_PER_STREAM = 128
NBUF = 8


def _make_gather(num_workers, streams_per_worker, n_rows, dim):
  mesh = plsc.VectorSubcoreMesh(core_axis_name="c", subcore_axis_name="s")
  rows_per_worker = streams_per_worker * ROWS_PER_STREAM

  @functools.partial(
      pl.kernel,
      out_type=jax.ShapeDtypeStruct((n_rows, dim), jnp.float32),
      mesh=mesh,
      scratch_types=[
          pltpu.VMEM((streams_per_worker, ROWS_PER_STREAM), jnp.int32),
          pltpu.VMEM((NBUF, ROWS_PER_STREAM, dim), jnp.float32),
          pltpu.SemaphoreType.DMA,
      ],
      compiler_params=pltpu.CompilerParams(use_tc_tiling_on_sc=False),
  )
  def gather_kernel(idx_hbm, table_hbm, out_hbm, idx_v, rows_v, gsem):
    num_cores = jax.lax.axis_size("c")
    wid = lax.axis_index("s") * num_cores + lax.axis_index("c")
    base = wid * rows_per_worker

    pltpu.sync_copy(idx_hbm.at[wid], idx_v)

    def start_gather(j, buf):
      pltpu.make_async_copy(
          table_hbm.at[idx_v.at[j]], rows_v.at[buf], gsem).start()

    def finish_gather(j, buf):
      pltpu.make_async_copy(
          table_hbm.at[idx_v.at[j]], rows_v.at[buf], gsem).wait()
      pltpu.sync_copy(
          rows_v.at[buf],
          out_hbm.at[pl.ds(base + j * ROWS_PER_STREAM, ROWS_PER_STREAM)])

    for b in range(NBUF):
      start_gather(b, b)

    n_blocks = streams_per_worker // NBUF

    def body(i, carry):
      for b in range(NBUF):
        j = i * NBUF + b
        finish_gather(j, b)
        start_gather(j + NBUF, b)
      return carry

    lax.fori_loop(0, n_blocks - 1, body, 0)

    for b in range(NBUF):
      finish_gather((n_blocks - 1) * NBUF + b, b)

  return gather_kernel


def _mlp_body(e_ref, w1t_ref, b1_ref, w2_ref, b2_ref, o_ref):
  x2 = e_ref[...]
  d = w1t_ref.shape[0]
  blk2 = x2.shape[0]
  for half in range(2):
    x = x2[:, half * d:(half + 1) * d]
    h = jnp.dot(x, w1t_ref[...], preferred_element_type=jnp.float32)
    h = h + b1_ref[...]
    h = h * 0.5 * (1.0 + lax.erf(h * 0.7071067811865476))
    o = lax.dot_general(w2_ref[...], h, (((1,), (1,)), ((), ())),
                        preferred_element_type=jnp.float32)
    o_ref[0, :, half * blk2:(half + 1) * blk2] = o + b2_ref[...]


def _mlp_t(e2, w1t, b1, w2, b2c, fields, batch, blk2):
  d = w2.shape[0]
  nb = (batch // 2) // blk2
  return pl.pallas_call(
      _mlp_body,
      grid=(fields, nb),
      in_specs=[
          pl.BlockSpec((blk2, 2 * d), lambda f, j: (f * nb + j, 0)),
          pl.BlockSpec((d, d), lambda f, j: (0, 0)),
          pl.BlockSpec((1, d), lambda f, j: (0, 0)),
          pl.BlockSpec((d, d), lambda f, j: (0, 0)),
          pl.BlockSpec((d, 1), lambda f, j: (0, 0)),
      ],
      out_specs=pl.BlockSpec((1, d, 2 * blk2), lambda f, j: (f, 0, j)),
      out_shape=jax.ShapeDtypeStruct((fields, d, batch), jnp.float32),
      compiler_params=pltpu.CompilerParams(
          dimension_semantics=("arbitrary", "arbitrary")),
  )(e2, w1t, b1, w2, b2c)


def kernel(action_ids, W_emb, W1, b1, W2, b2):
  batch, fields = action_ids.shape
  n_rows = batch * fields
  dim = W_emb.shape[1]

  info = plsc.get_sparse_core_info()
  num_workers = info.num_cores * info.num_subcores
  streams_per_worker = n_rows // (num_workers * ROWS_PER_STREAM)

  blk2 = 1024
  idx = action_ids.T.reshape(fields, batch // (2 * blk2), 2, blk2)
  idx = idx.transpose(0, 1, 3, 2)
  idx3 = idx.reshape(num_workers, streams_per_worker, ROWS_PER_STREAM)
  e = _make_gather(num_workers, streams_per_worker, n_rows, dim)(idx3, W_emb)
  e2 = e.reshape(n_rows // 2, 2 * dim)
  out = _mlp_t(e2, W1.T, b1.reshape(1, dim), W2, b2.reshape(dim, 1),
               fields, batch, blk2)
  return out.transpose(2, 0, 1)


# speedup vs baseline: 1.2424x; 1.0605x over previous
import functools

import jax
import jax.numpy as jnp
from jax import lax
from jax.experimental import pallas as pl
from jax.experimental.pallas import tpu as pltpu
from jax.experimental.pallas import tpu_sc as plsc

EMBED_DIM = 64
ROWS_PER_STREAM = 128
NBUF = 8


def _make_gather(num_workers, streams_per_worker, n_rows, dim):
  mesh = plsc.VectorSubcoreMesh(core_axis_name="c", subcore_axis_name="s")
  rows_per_worker = streams_per_worker * ROWS_PER_STREAM

  @functools.partial(
      pl.kernel,
      out_type=jax.ShapeDtypeStruct((n_rows, dim), jnp.float32),
      mesh=mesh,
      scratch_types=[
          pltpu.VMEM((streams_per_worker, ROWS_PER_STREAM), jnp.int32),
          pltpu.VMEM((NBUF, ROWS_PER_STREAM, dim), jnp.float32),
          pltpu.SemaphoreType.DMA,
      ],
      compiler_params=pltpu.CompilerParams(use_tc_tiling_on_sc=False),
  )
  def gather_kernel(idx_hbm, table_hbm, out_hbm, idx_v, rows_v, gsem):
    num_cores = jax.lax.axis_size("c")
    wid = lax.axis_index("s") * num_cores + lax.axis_index("c")
    base = wid * rows_per_worker

    pltpu.sync_copy(idx_hbm.at[wid], idx_v)

    def start_gather(j, buf):
      pltpu.make_async_copy(
          table_hbm.at[idx_v.at[j]], rows_v.at[buf], gsem).start()

    def finish_gather(j, buf):
      pltpu.make_async_copy(
          table_hbm.at[idx_v.at[j]], rows_v.at[buf], gsem).wait()
      pltpu.sync_copy(
          rows_v.at[buf],
          out_hbm.at[pl.ds(base + j * ROWS_PER_STREAM, ROWS_PER_STREAM)])

    for b in range(NBUF):
      start_gather(b, b)

    n_blocks = streams_per_worker // NBUF

    def body(i, carry):
      for b in range(NBUF):
        j = i * NBUF + b
        finish_gather(j, b)
        start_gather(j + NBUF, b)
      return carry

    lax.fori_loop(0, n_blocks - 1, body, 0)

    for b in range(NBUF):
      finish_gather((n_blocks - 1) * NBUF + b, b)

  return gather_kernel


def _transpose_table(wt, blkc=2048):
  d, rows = wt.shape

  def body(w_ref, o_ref):
    x = w_ref[...]
    o_ref[:, :d] = x.T
    o_ref[:, d:] = jnp.zeros((blkc, d), jnp.float32)

  return pl.pallas_call(
      body,
      grid=(pl.cdiv(rows, blkc),),
      in_specs=[pl.BlockSpec((d, blkc), lambda j: (0, j))],
      out_specs=pl.BlockSpec((blkc, 2 * d), lambda j: (j, 0)),
      out_shape=jax.ShapeDtypeStruct((rows, 2 * d), jnp.float32),
      compiler_params=pltpu.CompilerParams(
          dimension_semantics=("arbitrary",)),
  )(wt)


def _mlp_body(e_ref, w1t_ref, b1_ref, w2_ref, b2_ref, o_ref):
  x2 = e_ref[...]
  d = w1t_ref.shape[0]
  blk2 = x2.shape[0]
  for half in range(2):
    x = x2[:, half * d:(half + 1) * d]
    h = jnp.dot(x, w1t_ref[...], preferred_element_type=jnp.float32)
    h = h + b1_ref[...]
    h = h * 0.5 * (1.0 + lax.erf(h * 0.7071067811865476))
    o = lax.dot_general(w2_ref[...], h, (((1,), (1,)), ((), ())),
                        preferred_element_type=jnp.float32)
    o_ref[0, :, half * blk2:(half + 1) * blk2] = o + b2_ref[...]


def _mlp_t(e2, w1t, b1, w2, b2c, fields, batch, blk2):
  d = w2.shape[0]
  nb = (batch // 2) // blk2
  return pl.pallas_call(
      _mlp_body,
      grid=(fields, nb),
      in_specs=[
          pl.BlockSpec((blk2, 2 * d), lambda f, j: (f * nb + j, 0)),
          pl.BlockSpec((d, d), lambda f, j: (0, 0)),
          pl.BlockSpec((1, d), lambda f, j: (0, 0)),
          pl.BlockSpec((d, d), lambda f, j: (0, 0)),
          pl.BlockSpec((d, 1), lambda f, j: (0, 0)),
      ],
      out_specs=pl.BlockSpec((1, d, 2 * blk2), lambda f, j: (f, 0, j)),
      out_shape=jax.ShapeDtypeStruct((fields, d, batch), jnp.float32),
      compiler_params=pltpu.CompilerParams(
          dimension_semantics=("arbitrary", "arbitrary")),
  )(e2, w1t, b1, w2, b2c)


def kernel(action_ids, W_emb, W1, b1, W2, b2):
  batch, fields = action_ids.shape
  n_rows = batch * fields
  dim = W_emb.shape[1]

  info = plsc.get_sparse_core_info()
  num_workers = info.num_cores * info.num_subcores
  streams_per_worker = n_rows // (num_workers * ROWS_PER_STREAM)

  blk2 = 1024
  idx = action_ids.T.reshape(fields, batch // (2 * blk2), 2, blk2)
  idx = idx.transpose(0, 1, 3, 2) * 2
  idx3 = idx.reshape(num_workers, streams_per_worker, ROWS_PER_STREAM)
  tbl = _transpose_table(W_emb.T).reshape(2 * W_emb.shape[0], dim)
  e = _make_gather(num_workers, streams_per_worker, n_rows, dim)(idx3, tbl)
  e2 = e.reshape(n_rows // 2, 2 * dim)
  out = _mlp_t(e2, W1.T, b1.reshape(1, dim), W2, b2.reshape(dim, 1),
               fields, batch, blk2)
  return out.transpose(2, 0, 1)


# speedup vs baseline: 1.2434x; 1.0008x over previous
import functools

import jax
import jax.numpy as jnp
from jax import lax
from jax.experimental import pallas as pl
from jax.experimental.pallas import tpu as pltpu
from jax.experimental.pallas import tpu_sc as plsc

EMBED_DIM = 64
ROWS_PER_STREAM = 128
NBUF = 8


def _make_gather(num_workers, streams_per_worker, n_rows, dim):
  mesh = plsc.VectorSubcoreMesh(core_axis_name="c", subcore_axis_name="s")
  rows_per_worker = streams_per_worker * ROWS_PER_STREAM

  @functools.partial(
      pl.kernel,
      out_type=jax.ShapeDtypeStruct((n_rows, dim), jnp.float32),
      mesh=mesh,
      scratch_types=[
          pltpu.VMEM((streams_per_worker, ROWS_PER_STREAM), jnp.int32),
          pltpu.VMEM((NBUF, ROWS_PER_STREAM, dim), jnp.float32),
          pltpu.SemaphoreType.DMA,
      ],
      compiler_params=pltpu.CompilerParams(use_tc_tiling_on_sc=False),
  )
  def gather_kernel(idx_hbm, table_hbm, out_hbm, idx_v, rows_v, gsem):
    num_cores = jax.lax.axis_size("c")
    wid = lax.axis_index("s") * num_cores + lax.axis_index("c")
    base = wid * rows_per_worker

    pltpu.sync_copy(idx_hbm.at[wid], idx_v)

    def start_gather(j, buf):
      pltpu.make_async_copy(
          table_hbm.at[idx_v.at[j]], rows_v.at[buf], gsem).start()

    def finish_gather(j, buf):
      pltpu.make_async_copy(
          table_hbm.at[idx_v.at[j]], rows_v.at[buf], gsem).wait()
      pltpu.sync_copy(
          rows_v.at[buf],
          out_hbm.at[pl.ds(base + j * ROWS_PER_STREAM, ROWS_PER_STREAM)])

    for b in range(NBUF):
      start_gather(b, b)

    n_blocks = streams_per_worker // NBUF

    def body(i, carry):
      for b in range(NBUF):
        j = i * NBUF + b
        finish_gather(j, b)
        start_gather(j + NBUF, b)
      return carry

    lax.fori_loop(0, n_blocks - 1, body, 0)

    for b in range(NBUF):
      finish_gather((n_blocks - 1) * NBUF + b, b)

  return gather_kernel


def _transpose_table(wt, blkc=2048):
  d, rows = wt.shape

  def body(w_ref, o_ref):
    o_ref[:, :d] = w_ref[...].T

  return pl.pallas_call(
      body,
      grid=(pl.cdiv(rows, blkc),),
      in_specs=[pl.BlockSpec((d, blkc), lambda j: (0, j))],
      out_specs=pl.BlockSpec((blkc, 2 * d), lambda j: (j, 0)),
      out_shape=jax.ShapeDtypeStruct((rows, 2 * d), jnp.float32),
      compiler_params=pltpu.CompilerParams(
          dimension_semantics=("arbitrary",)),
  )(wt)


def _mlp_body(e_ref, w1t_ref, b1_ref, w2_ref, b2_ref, o_ref):
  x2 = e_ref[...]
  d = w1t_ref.shape[0]
  blk2 = x2.shape[0]
  for half in range(2):
    x = x2[:, half * d:(half + 1) * d]
    h = jnp.dot(x, w1t_ref[...], preferred_element_type=jnp.float32)
    h = h + b1_ref[...]
    h = h * 0.5 * (1.0 + lax.erf(h * 0.7071067811865476))
    o = lax.dot_general(w2_ref[...], h, (((1,), (1,)), ((), ())),
                        preferred_element_type=jnp.float32)
    o_ref[0, :, half * blk2:(half + 1) * blk2] = o + b2_ref[...]


def _mlp_t(e2, w1t, b1, w2, b2c, fields, batch, blk2):
  d = w2.shape[0]
  nb = (batch // 2) // blk2
  return pl.pallas_call(
      _mlp_body,
      grid=(fields, nb),
      in_specs=[
          pl.BlockSpec((blk2, 2 * d), lambda f, j: (f * nb + j, 0)),
          pl.BlockSpec((d, d), lambda f, j: (0, 0)),
          pl.BlockSpec((1, d), lambda f, j: (0, 0)),
          pl.BlockSpec((d, d), lambda f, j: (0, 0)),
          pl.BlockSpec((d, 1), lambda f, j: (0, 0)),
      ],
      out_specs=pl.BlockSpec((1, d, 2 * blk2), lambda f, j: (f, 0, j)),
      out_shape=jax.ShapeDtypeStruct((fields, d, batch), jnp.float32),
      compiler_params=pltpu.CompilerParams(
          dimension_semantics=("arbitrary", "arbitrary")),
  )(e2, w1t, b1, w2, b2c)


def kernel(action_ids, W_emb, W1, b1, W2, b2):
  batch, fields = action_ids.shape
  n_rows = batch * fields
  dim = W_emb.shape[1]

  info = plsc.get_sparse_core_info()
  num_workers = info.num_cores * info.num_subcores
  streams_per_worker = n_rows // (num_workers * ROWS_PER_STREAM)

  blk2 = 1024
  idx = action_ids.T.reshape(fields, batch // (2 * blk2), 2, blk2)
  idx = idx.transpose(0, 1, 3, 2) * 2
  idx3 = idx.reshape(num_workers, streams_per_worker, ROWS_PER_STREAM)
  tbl = _transpose_table(W_emb.T).reshape(2 * W_emb.shape[0], dim)
  e = _make_gather(num_workers, streams_per_worker, n_rows, dim)(idx3, tbl)
  e2 = e.reshape(n_rows // 2, 2 * dim)
  out = _mlp_t(e2, W1.T, b1.reshape(1, dim), W2, b2.reshape(dim, 1),
               fields, batch, blk2)
  return out.transpose(2, 0, 1)
